# baseline (speedup 1.0000x reference)
import functools

import jax
import jax.numpy as jnp
from jax import lax
from jax.experimental import pallas as pl
from jax.experimental.pallas import tpu as pltpu
from jax.experimental.pallas import tpu_sc as plsc

B, N, M, C1, C2, CO = 4, 8192, 1024, 32, 64, 128
TN = 512
TNC = 2048
NSC = 32


def _nn3_kernel(u_ref, k_ref, oi_ref, ow_ref):
    u = u_ref[0]
    kpts = k_ref[0]

    a = jnp.concatenate([-2.0 * u, jnp.ones((TN, 1), jnp.float32)], axis=1)
    kn2 = jnp.sum(kpts * kpts, axis=1, keepdims=True)
    kb = jnp.concatenate([kpts, kn2], axis=1)
    d2 = lax.dot_general(a, kb, (((1,), (1,)), ((), ())),
                         preferred_element_type=jnp.float32,
                         precision=lax.Precision.HIGHEST)
    un2 = jnp.sum(u * u, axis=1)

    iota = lax.broadcasted_iota(jnp.int32, (TN, M), 1).astype(jnp.float32)

    vals = []
    idxs = []
    for _ in range(3):
        mval = jnp.min(d2, axis=1, keepdims=True)
        hit = d2 == mval
        ji = jnp.min(jnp.where(hit, iota, float(M)), axis=1)
        vals.append(mval[:, 0] + un2)
        idxs.append(ji)
        d2 = jnp.where(iota == ji[:, None], jnp.inf, d2)

    rs = [1.0 / (jnp.sqrt(jnp.maximum(v, 0.0)) + 1e-8) for v in vals]
    norm = (rs[0] + rs[1]) + rs[2]

    oi_ref[0] = jnp.stack([ji.astype(jnp.int32) for ji in idxs], axis=0)
    ow_ref[0] = jnp.stack([r / norm for r in rs], axis=0)


def _three_nn(unknown, known):
    nb = unknown.shape[0]
    return pl.pallas_call(
        _nn3_kernel,
        grid=(nb, N // TN),
        in_specs=[
            pl.BlockSpec((1, TN, 3), lambda bb, i: (bb, i, 0)),
            pl.BlockSpec((1, M, 3), lambda bb, i: (bb, 0, 0)),
        ],
        out_specs=[
            pl.BlockSpec((1, 3, TN), lambda bb, i: (bb, 0, i)),
            pl.BlockSpec((1, 3, TN), lambda bb, i: (bb, 0, i)),
        ],
        out_shape=[
            jax.ShapeDtypeStruct((nb, 3, N), jnp.int32),
            jax.ShapeDtypeStruct((nb, 3, N), jnp.float32),
        ],
    )(unknown, known)


def _sc_interpolate(kft, idx3, w3):
    nb = kft.shape[0]
    q = (nb * N) // NSC
    qc = min(q, 512)
    nch = q // qc
    grp = qc // 16
    wpb = N // q

    def body(kft_ref, idx_ref, w_ref, out_ref, table_v, idx_v, w_v, acc_v):
        wid = lax.axis_index("s") * 2 + lax.axis_index("c")
        b = wid // wpb
        qo = (wid % wpb) * q

        pltpu.sync_copy(kft_ref.at[b], table_v)
        pltpu.sync_copy(idx_ref.at[b, :, pl.ds(qo, q)], idx_v)
        pltpu.sync_copy(w_ref.at[b, :, pl.ds(qo, q)], w_v)

        for ch in range(nch):
            @plsc.parallel_loop(0, grp, unroll=2)
            def group(g):
                base = ch * qc + g * 16
                s0 = idx_v[0, pl.ds(base, 16)]
                s1 = idx_v[1, pl.ds(base, 16)]
                s2 = idx_v[2, pl.ds(base, 16)]
                w0 = w_v[0, pl.ds(base, 16)]
                w1 = w_v[1, pl.ds(base, 16)]
                w2 = w_v[2, pl.ds(base, 16)]
                for c in range(C2):
                    g0 = plsc.load_gather(table_v, [s0 + c * M])
                    g1 = plsc.load_gather(table_v, [s1 + c * M])
                    g2 = plsc.load_gather(table_v, [s2 + c * M])
                    acc = (g0 * w0 + g1 * w1) + g2 * w2
                    acc_v[c, pl.ds(g * 16, 16)] = acc

            pltpu.sync_copy(acc_v, out_ref.at[b, :, pl.ds(qo + ch * qc, qc)])

    mesh = plsc.VectorSubcoreMesh(core_axis_name="c", subcore_axis_name="s",
                                  num_cores=2, num_subcores=16)
    f = functools.partial(
        pl.kernel,
        out_type=jax.ShapeDtypeStruct((nb, C2, N), jnp.float32),
        mesh=mesh,
        compiler_params=pltpu.CompilerParams(needs_layout_passes=False),
        scratch_types=[
            pltpu.VMEM((M * C2,), jnp.float32),
            pltpu.VMEM((3, q), jnp.int32),
            pltpu.VMEM((3, q), jnp.float32),
            pltpu.VMEM((C2, qc), jnp.float32),
        ],
    )(body)
    return f(kft, idx3, w3)


def _mlp_kernel(if_ref, uf_ref, w_ref, b_ref, o_ref):
    w = w_ref[...]
    out = jnp.dot(w[:, :C2], if_ref[0], preferred_element_type=jnp.float32)
    out = out + jnp.dot(w[:, C2:], uf_ref[0],
                        preferred_element_type=jnp.float32)
    out = out + b_ref[...]
    o_ref[0] = jnp.maximum(out, 0.0)


def _mlp(interp, unknow_feats, W, b):
    nb = interp.shape[0]
    return pl.pallas_call(
        _mlp_kernel,
        grid=(nb, N // TNC),
        in_specs=[
            pl.BlockSpec((1, C2, TNC), lambda bb, i: (bb, 0, i)),
            pl.BlockSpec((1, C1, TNC), lambda bb, i: (bb, 0, i)),
            pl.BlockSpec((CO, C1 + C2), lambda bb, i: (0, 0)),
            pl.BlockSpec((CO, 1), lambda bb, i: (0, 0)),
        ],
        out_specs=pl.BlockSpec((1, CO, TNC), lambda bb, i: (bb, 0, i)),
        out_shape=jax.ShapeDtypeStruct((nb, CO, N), jnp.float32),
    )(interp, unknow_feats, W, b.reshape(CO, 1))


@jax.jit
def kernel(unknown, known, unknow_feats, known_feats, W, b):
    idx3, w3 = _three_nn(unknown, known)
    kft = known_feats.reshape(B, C2 * M)
    interp = _sc_interpolate(kft, idx3, w3)
    return _mlp(interp, unknow_feats, W, b)


# speedup vs baseline: 1.2067x; 1.2067x over previous
import functools

import jax
import jax.numpy as jnp
from jax import lax
from jax.experimental import pallas as pl
from jax.experimental.pallas import tpu as pltpu
from jax.experimental.pallas import tpu_sc as plsc

B, N, M, C1, C2, CO = 4, 8192, 1024, 32, 64, 128
TN = 512
TNC = 2048
NSC = 32


def _nn3_kernel(u_ref, k_ref, oi_ref, ow_ref):
    u = u_ref[0]
    kpts = k_ref[0]

    e0 = (u[:, 0:1] - kpts[:, 0][None, :]) ** 2
    e1 = (u[:, 1:2] - kpts[:, 1][None, :]) ** 2
    e2 = (u[:, 2:3] - kpts[:, 2][None, :]) ** 2
    d2 = (e0 + e1) + e2

    iota = lax.broadcasted_iota(jnp.int32, (TN, M), 1).astype(jnp.float32)

    vals = []
    idxs = []
    for _ in range(3):
        mval = jnp.min(d2, axis=1, keepdims=True)
        hit = d2 == mval
        ji = jnp.min(jnp.where(hit, iota, float(M)), axis=1)
        vals.append(mval[:, 0])
        idxs.append(ji)
        d2 = jnp.where(iota == ji[:, None], jnp.inf, d2)

    rs = [1.0 / (jnp.sqrt(jnp.maximum(v, 0.0)) + 1e-8) for v in vals]
    norm = (rs[0] + rs[1]) + rs[2]

    oi_ref[0] = jnp.stack([ji.astype(jnp.int32) for ji in idxs], axis=0)
    ow_ref[0] = jnp.stack([r / norm for r in rs], axis=0)


def _three_nn(unknown, known):
    nb = unknown.shape[0]
    return pl.pallas_call(
        _nn3_kernel,
        grid=(nb, N // TN),
        in_specs=[
            pl.BlockSpec((1, TN, 3), lambda bb, i: (bb, i, 0)),
            pl.BlockSpec((1, M, 3), lambda bb, i: (bb, 0, 0)),
        ],
        out_specs=[
            pl.BlockSpec((1, 3, TN), lambda bb, i: (bb, 0, i)),
            pl.BlockSpec((1, 3, TN), lambda bb, i: (bb, 0, i)),
        ],
        out_shape=[
            jax.ShapeDtypeStruct((nb, 3, N), jnp.int32),
            jax.ShapeDtypeStruct((nb, 3, N), jnp.float32),
        ],
    )(unknown, known)


def _sc_interpolate(kft, idx3, w3):
    nb = kft.shape[0]
    q = (nb * N) // NSC
    qc = min(q, 512)
    nch = q // qc
    grp = qc // 16
    wpb = N // q

    def body(kft_ref, idx_ref, w_ref, out_ref, table_v, idx_v, w_v, acc_v):
        wid = lax.axis_index("s") * 2 + lax.axis_index("c")
        b = wid // wpb
        qo = (wid % wpb) * q

        pltpu.sync_copy(kft_ref.at[b], table_v)
        pltpu.sync_copy(idx_ref.at[b, :, pl.ds(qo, q)], idx_v)
        pltpu.sync_copy(w_ref.at[b, :, pl.ds(qo, q)], w_v)

        for ch in range(nch):
            @plsc.parallel_loop(0, grp, unroll=2)
            def group(g):
                base = ch * qc + g * 16
                s0 = idx_v[0, pl.ds(base, 16)]
                s1 = idx_v[1, pl.ds(base, 16)]
                s2 = idx_v[2, pl.ds(base, 16)]
                w0 = w_v[0, pl.ds(base, 16)]
                w1 = w_v[1, pl.ds(base, 16)]
                w2 = w_v[2, pl.ds(base, 16)]
                for c in range(C2):
                    g0 = plsc.load_gather(table_v, [s0 + c * M])
                    g1 = plsc.load_gather(table_v, [s1 + c * M])
                    g2 = plsc.load_gather(table_v, [s2 + c * M])
                    acc = (g0 * w0 + g1 * w1) + g2 * w2
                    acc_v[c, pl.ds(g * 16, 16)] = acc

            pltpu.sync_copy(acc_v, out_ref.at[b, :, pl.ds(qo + ch * qc, qc)])

    mesh = plsc.VectorSubcoreMesh(core_axis_name="c", subcore_axis_name="s",
                                  num_cores=2, num_subcores=16)
    f = functools.partial(
        pl.kernel,
        out_type=jax.ShapeDtypeStruct((nb, C2, N), jnp.float32),
        mesh=mesh,
        compiler_params=pltpu.CompilerParams(needs_layout_passes=False),
        scratch_types=[
            pltpu.VMEM((M * C2,), jnp.float32),
            pltpu.VMEM((3, q), jnp.int32),
            pltpu.VMEM((3, q), jnp.float32),
            pltpu.VMEM((C2, qc), jnp.float32),
        ],
    )(body)
    return f(kft, idx3, w3)


def _mlp_kernel(if_ref, uf_ref, w_ref, b_ref, o_ref):
    w = w_ref[...]
    out = jnp.dot(w[:, :C2], if_ref[0], preferred_element_type=jnp.float32)
    out = out + jnp.dot(w[:, C2:], uf_ref[0],
                        preferred_element_type=jnp.float32)
    out = out + b_ref[...]
    o_ref[0] = jnp.maximum(out, 0.0)


def _mlp(interp, unknow_feats, W, b):
    nb = interp.shape[0]
    return pl.pallas_call(
        _mlp_kernel,
        grid=(nb, N // TNC),
        in_specs=[
            pl.BlockSpec((1, C2, TNC), lambda bb, i: (bb, 0, i)),
            pl.BlockSpec((1, C1, TNC), lambda bb, i: (bb, 0, i)),
            pl.BlockSpec((CO, C1 + C2), lambda bb, i: (0, 0)),
            pl.BlockSpec((CO, 1), lambda bb, i: (0, 0)),
        ],
        out_specs=pl.BlockSpec((1, CO, TNC), lambda bb, i: (bb, 0, i)),
        out_shape=jax.ShapeDtypeStruct((nb, CO, N), jnp.float32),
    )(interp, unknow_feats, W, b.reshape(CO, 1))


@jax.jit
def kernel(unknown, known, unknow_feats, known_feats, W, b):
    idx3, w3 = _three_nn(unknown, known)
    kft = known_feats.reshape(B, C2 * M)
    interp = _sc_interpolate(kft, idx3, w3)
    return _mlp(interp, unknow_feats, W, b)


# speedup vs baseline: 1.2557x; 1.0405x over previous
import functools

import jax
import jax.numpy as jnp
from jax import lax
from jax.experimental import pallas as pl
from jax.experimental.pallas import tpu as pltpu
from jax.experimental.pallas import tpu_sc as plsc

B, N, M, C1, C2, CO = 4, 8192, 1024, 32, 64, 128
TN = 1024
TNC = 2048
NSC = 32


def _nn3_kernel(u_ref, k_ref, oi_ref, ow_ref):
    u = u_ref[0]
    kpts = k_ref[0]

    e0 = (u[:, 0:1] - kpts[:, 0][None, :]) ** 2
    e1 = (u[:, 1:2] - kpts[:, 1][None, :]) ** 2
    e2 = (u[:, 2:3] - kpts[:, 2][None, :]) ** 2
    d2 = (e0 + e1) + e2

    iota = lax.broadcasted_iota(jnp.int32, (TN, M), 1).astype(jnp.float32)

    vals = []
    idxs = []
    for _ in range(3):
        mval = jnp.min(d2, axis=1, keepdims=True)
        hit = d2 == mval
        ji = jnp.min(jnp.where(hit, iota, float(M)), axis=1)
        vals.append(mval[:, 0])
        idxs.append(ji)
        d2 = jnp.where(iota == ji[:, None], jnp.inf, d2)

    rs = [1.0 / (jnp.sqrt(jnp.maximum(v, 0.0)) + 1e-8) for v in vals]
    norm = (rs[0] + rs[1]) + rs[2]

    oi_ref[0] = jnp.stack([ji.astype(jnp.int32) for ji in idxs], axis=0)
    ow_ref[0] = jnp.stack([r / norm for r in rs], axis=0)


def _three_nn(unknown, known):
    nb = unknown.shape[0]
    return pl.pallas_call(
        _nn3_kernel,
        grid=(nb, N // TN),
        in_specs=[
            pl.BlockSpec((1, TN, 3), lambda bb, i: (bb, i, 0)),
            pl.BlockSpec((1, M, 3), lambda bb, i: (bb, 0, 0)),
        ],
        out_specs=[
            pl.BlockSpec((1, 3, TN), lambda bb, i: (bb, 0, i)),
            pl.BlockSpec((1, 3, TN), lambda bb, i: (bb, 0, i)),
        ],
        out_shape=[
            jax.ShapeDtypeStruct((nb, 3, N), jnp.int32),
            jax.ShapeDtypeStruct((nb, 3, N), jnp.float32),
        ],
    )(unknown, known)


def _sc_interpolate(kft, idx3, w3):
    nb = kft.shape[0]
    q = (nb * N) // NSC
    qc = min(q, 512)
    nch = q // qc
    grp = qc // 16
    wpb = N // q

    def body(kft_ref, idx_ref, w_ref, out_ref, table_v, idx_v, w_v, acc_v):
        wid = lax.axis_index("s") * 2 + lax.axis_index("c")
        b = wid // wpb
        qo = (wid % wpb) * q

        pltpu.sync_copy(kft_ref.at[b], table_v)
        pltpu.sync_copy(idx_ref.at[b, :, pl.ds(qo, q)], idx_v)
        pltpu.sync_copy(w_ref.at[b, :, pl.ds(qo, q)], w_v)

        for ch in range(nch):
            @plsc.parallel_loop(0, grp, unroll=2)
            def group(g):
                base = ch * qc + g * 16
                s0 = idx_v[0, pl.ds(base, 16)]
                s1 = idx_v[1, pl.ds(base, 16)]
                s2 = idx_v[2, pl.ds(base, 16)]
                w0 = w_v[0, pl.ds(base, 16)]
                w1 = w_v[1, pl.ds(base, 16)]
                w2 = w_v[2, pl.ds(base, 16)]
                for c in range(C2):
                    g0 = plsc.load_gather(table_v, [s0 + c * M])
                    g1 = plsc.load_gather(table_v, [s1 + c * M])
                    g2 = plsc.load_gather(table_v, [s2 + c * M])
                    acc = (g0 * w0 + g1 * w1) + g2 * w2
                    acc_v[c, pl.ds(g * 16, 16)] = acc

            pltpu.sync_copy(acc_v, out_ref.at[b, :, pl.ds(qo + ch * qc, qc)])

    mesh = plsc.VectorSubcoreMesh(core_axis_name="c", subcore_axis_name="s",
                                  num_cores=2, num_subcores=16)
    f = functools.partial(
        pl.kernel,
        out_type=jax.ShapeDtypeStruct((nb, C2, N), jnp.float32),
        mesh=mesh,
        compiler_params=pltpu.CompilerParams(needs_layout_passes=False),
        scratch_types=[
            pltpu.VMEM((M * C2,), jnp.float32),
            pltpu.VMEM((3, q), jnp.int32),
            pltpu.VMEM((3, q), jnp.float32),
            pltpu.VMEM((C2, qc), jnp.float32),
        ],
    )(body)
    return f(kft, idx3, w3)


def _mlp_kernel(if_ref, uf_ref, w_ref, b_ref, o_ref):
    w = w_ref[...]
    out = jnp.dot(w[:, :C2], if_ref[0], preferred_element_type=jnp.float32)
    out = out + jnp.dot(w[:, C2:], uf_ref[0],
                        preferred_element_type=jnp.float32)
    out = out + b_ref[...]
    o_ref[0] = jnp.maximum(out, 0.0)


def _mlp(interp, unknow_feats, W, b):
    nb = interp.shape[0]
    return pl.pallas_call(
        _mlp_kernel,
        grid=(nb, N // TNC),
        in_specs=[
            pl.BlockSpec((1, C2, TNC), lambda bb, i: (bb, 0, i)),
            pl.BlockSpec((1, C1, TNC), lambda bb, i: (bb, 0, i)),
            pl.BlockSpec((CO, C1 + C2), lambda bb, i: (0, 0)),
            pl.BlockSpec((CO, 1), lambda bb, i: (0, 0)),
        ],
        out_specs=pl.BlockSpec((1, CO, TNC), lambda bb, i: (bb, 0, i)),
        out_shape=jax.ShapeDtypeStruct((nb, CO, N), jnp.float32),
    )(interp, unknow_feats, W, b.reshape(CO, 1))


@jax.jit
def kernel(unknown, known, unknow_feats, known_feats, W, b):
    idx3, w3 = _three_nn(unknown, known)
    kft = known_feats.reshape(B, C2 * M)
    interp = _sc_interpolate(kft, idx3, w3)
    return _mlp(interp, unknow_feats, W, b)
